# baseline (speedup 1.0000x reference)
import functools

import jax
import jax.numpy as jnp
import numpy as np
from jax import lax
from jax.experimental import pallas as pl
from jax.experimental.pallas import tpu as pltpu
from jax.experimental.pallas import tpu_sc as plsc

N = 8192
E = 4096
D = 128
DH = 128
H = 4
HD = DH // H
S = 32

_SCALE = float(1.0 / np.sqrt(HD))
_SCALE2 = float(1.0 / (np.sqrt(HD) * np.log(2.0)))
_NEG = -1e9
_EPS = 1e-5

BQ = 512
NB = E // BQ


def _sc_gather(x, idx):
    B = idx.shape[0]
    info = plsc.get_sparse_core_info()
    nw = info.num_cores * info.num_subcores
    b_per_w = B // nw
    ch = 128
    n_ch = b_per_w // ch
    mesh = plsc.VectorSubcoreMesh(core_axis_name="c", subcore_axis_name="s")

    @functools.partial(
        pl.kernel,
        mesh=mesh,
        out_type=jax.ShapeDtypeStruct((B, D), jnp.float32),
        scratch_types=[
            pltpu.VMEM((b_per_w,), jnp.int32),
            pltpu.VMEM((b_per_w, D), jnp.float32),
            pltpu.SemaphoreType.DMA,
        ],
    )
    def gather_k(table_hbm, idx_hbm, out_hbm, idx_v, rows_v, sem):
        wid = lax.axis_index("s") * info.num_cores + lax.axis_index("c")
        base = wid * b_per_w
        pltpu.sync_copy(idx_hbm.at[pl.ds(base, b_per_w)], idx_v)
        copies = []
        for j in range(n_ch):
            copies.append(
                pltpu.async_copy(
                    table_hbm.at[idx_v.at[pl.ds(j * ch, ch)]],
                    rows_v.at[pl.ds(j * ch, ch)],
                    sem,
                )
            )
        for c in copies:
            c.wait()
        pltpu.sync_copy(rows_v, out_hbm.at[pl.ds(base, b_per_w)])

    return gather_k(x, idx)


def _fused_body(
    g_ref, we_ref, wq_ref, wk_ref, wv_ref, wo_ref, bsel_ref,
    sc_ref, dc_ref, sr_ref, dr_ref, g1_ref, b1_ref,
    seeds_ref, wq2_ref, wk2_ref, wv2_ref, wo2_ref, g2_ref, b2_ref,
    p_ref,
    e_s, qa_s, ka_s, va_s, h_s, vbar_s,
):
    i = pl.program_id(0)

    @pl.when(i == 0)
    def _proj():
        xs = g_ref[0:E, :]
        xd = g_ref[E : 2 * E, :]
        e = jnp.dot(xs, we_ref[0:D, :], preferred_element_type=jnp.float32)
        e = e + jnp.dot(xd, we_ref[D : 2 * D, :], preferred_element_type=jnp.float32)
        e_s[...] = e
        q = jnp.dot(e, wq_ref[...], preferred_element_type=jnp.float32) * _SCALE2
        k = jnp.dot(e, wk_ref[...], preferred_element_type=jnp.float32)
        v = jnp.dot(e, wv_ref[...], preferred_element_type=jnp.float32)
        vbar_s[...] = jnp.mean(v, axis=0, keepdims=True)
        qn2 = jnp.dot(q * q, bsel_ref[...], preferred_element_type=jnp.float32)
        kn2 = jnp.dot(k * k, bsel_ref[...], preferred_element_type=jnp.float32)
        km = jnp.max(kn2, axis=0, keepdims=True)
        mh = jnp.sqrt(qn2 * km)
        ones = jnp.ones((E, 1), jnp.float32)
        zpad = jnp.zeros((E, 2 * HD - (HD + 1)), jnp.float32)
        for hh in range(H):
            sl = slice(hh * HD, (hh + 1) * HD)
            qa_s[hh] = jnp.concatenate([q[:, sl], mh[:, hh : hh + 1], zpad], axis=1)
            ka_s[hh] = jnp.concatenate([k[:, sl], -ones, zpad], axis=1)
            va_s[hh] = jnp.concatenate([v[:, sl], ones, zpad], axis=1).astype(
                jnp.bfloat16
            )

    sc = sc_ref[...]
    dc = dc_ref[...]
    sr = sr_ref[...]
    dr = dr_ref[...]
    adj = (sc == sr) | (dc == dr) | (sc == dr) | (dc == sr)
    row_ids = lax.broadcasted_iota(jnp.int32, (BQ, 1), 0) + i * BQ
    col_ids = lax.broadcasted_iota(jnp.int32, (1, E), 1)
    adjf = ((row_ids != col_ids) & adj).astype(jnp.bfloat16)

    outs = []
    for hh in range(H):
        qa = qa_s[hh, pl.ds(i * BQ, BQ), :]
        s = jax.lax.dot_general(
            qa, ka_s[hh], (((1,), (1,)), ((), ())), preferred_element_type=jnp.float32
        )
        p = jnp.exp2(s).astype(jnp.bfloat16) * adjf
        oa = jax.lax.dot_general(
            p, va_s[hh], (((1,), (0,)), ((), ())), preferred_element_type=jnp.float32
        )
        z = oa[:, HD : HD + 1]
        zs = jnp.where(z > 0.0, z, 1.0)
        o = oa[:, 0:HD] / zs
        outs.append(jnp.where(z > 0.0, o, vbar_s[:, hh * HD : (hh + 1) * HD]))
    o = jnp.concatenate(outs, axis=-1)
    hb = e_s[pl.ds(i * BQ, BQ), :] + jnp.dot(
        o, wo_ref[...], preferred_element_type=jnp.float32
    )
    mu = jnp.mean(hb, axis=-1, keepdims=True)
    var = jnp.mean((hb - mu) ** 2, axis=-1, keepdims=True)
    h_s[pl.ds(i * BQ, BQ), :] = (hb - mu) / jnp.sqrt(var + _EPS) * g1_ref[
        ...
    ] + b1_ref[...]

    @pl.when(i == NB - 1)
    def _pma():
        seeds = seeds_ref[...]
        hm = h_s[...]
        q2 = jnp.dot(seeds, wq2_ref[...], preferred_element_type=jnp.float32) * _SCALE
        k2 = jnp.dot(hm, wk2_ref[...], preferred_element_type=jnp.float32)
        v2 = jnp.dot(hm, wv2_ref[...], preferred_element_type=jnp.float32)
        outs2 = []
        for hh in range(H):
            qh = q2[:, hh * HD : (hh + 1) * HD]
            kh = k2[:, hh * HD : (hh + 1) * HD]
            vh = v2[:, hh * HD : (hh + 1) * HD]
            s = jax.lax.dot_general(
                qh, kh, (((1,), (1,)), ((), ())), preferred_element_type=jnp.float32
            )
            m = jnp.max(s, axis=-1, keepdims=True)
            p = jnp.exp(s - m)
            z = jnp.sum(p, axis=-1, keepdims=True)
            outs2.append(jnp.dot(p, vh, preferred_element_type=jnp.float32) / z)
        o2 = jnp.concatenate(outs2, axis=-1)
        pb = seeds + jnp.dot(o2, wo2_ref[...], preferred_element_type=jnp.float32)
        mu2 = jnp.mean(pb, axis=-1, keepdims=True)
        var2 = jnp.mean((pb - mu2) ** 2, axis=-1, keepdims=True)
        p_ref[...] = (pb - mu2) / jnp.sqrt(var2 + _EPS) * g2_ref[...] + b2_ref[...]


def _tc_pipeline(g, edge_index, W_e, Wq, Wk, Wv, Wo, seeds,
                 Wq2, Wk2, Wv2, Wo2, g1, b1, g2, b2, interpret=False):
    f32 = jnp.float32
    src_c = edge_index[0].reshape(E, 1)
    dst_c = edge_index[1].reshape(E, 1)
    src_r = edge_index[0].reshape(1, E)
    dst_r = edge_index[1].reshape(1, E)
    bsel = jnp.asarray(
        np.repeat(np.eye(H, dtype=np.float32), HD, axis=0)
    )
    bsel = jnp.pad(bsel, ((0, 0), (0, DH - H)))
    full = lambda shape: pl.BlockSpec(shape, lambda i: (0, 0))
    p = pl.pallas_call(
        _fused_body,
        grid=(NB,),
        in_specs=[
            full((2 * E, D)),
            full((2 * D, DH)),
            full((DH, DH)),
            full((DH, DH)),
            full((DH, DH)),
            full((DH, DH)),
            full((DH, DH)),
            pl.BlockSpec((BQ, 1), lambda i: (i, 0)),
            pl.BlockSpec((BQ, 1), lambda i: (i, 0)),
            full((1, E)),
            full((1, E)),
            full((1, DH)),
            full((1, DH)),
            full((S, DH)),
            full((DH, DH)),
            full((DH, DH)),
            full((DH, DH)),
            full((DH, DH)),
            full((1, DH)),
            full((1, DH)),
        ],
        out_specs=pl.BlockSpec((S, DH), lambda i: (0, 0)),
        out_shape=jax.ShapeDtypeStruct((S, DH), f32),
        scratch_shapes=[
            pltpu.VMEM((E, DH), f32),
            pltpu.VMEM((H, E, 2 * HD), f32),
            pltpu.VMEM((H, E, 2 * HD), f32),
            pltpu.VMEM((H, E, 2 * HD), jnp.bfloat16),
            pltpu.VMEM((E, DH), f32),
            pltpu.VMEM((1, DH), f32),
        ],
        interpret=interpret,
    )(
        g, W_e, Wq, Wk, Wv, Wo, bsel, src_c, dst_c, src_r, dst_r,
        g1.reshape(1, DH), b1.reshape(1, DH), seeds,
        Wq2, Wk2, Wv2, Wo2, g2.reshape(1, DH), b2.reshape(1, DH),
    )
    return p


def kernel(x, edge_index, W_e, Wq, Wk, Wv, Wo, seeds, Wq2, Wk2, Wv2, Wo2, g1, b1, g2, b2):
    idx = edge_index.reshape(2 * E)
    g = _sc_gather(x, idx)
    return _tc_pipeline(g, edge_index, W_e, Wq, Wk, Wv, Wo, seeds,
                        Wq2, Wk2, Wv2, Wo2, g1, b1, g2, b2)


# speedup vs baseline: 1.0399x; 1.0399x over previous
import functools

import jax
import jax.numpy as jnp
import numpy as np
from jax import lax
from jax.experimental import pallas as pl
from jax.experimental.pallas import tpu as pltpu
from jax.experimental.pallas import tpu_sc as plsc

N = 8192
E = 4096
D = 128
DH = 128
H = 4
HD = DH // H
S = 32

_SCALE = float(1.0 / np.sqrt(HD))
_SCALE2 = float(1.0 / (np.sqrt(HD) * np.log(2.0)))
_NEG = -1e9
_EPS = 1e-5

BQ = 512
NB = E // BQ
KB = 256
KN = E // KB


def _sc_gather(x, idx):
    B = idx.shape[0]
    info = plsc.get_sparse_core_info()
    nw = info.num_cores * info.num_subcores
    b_per_w = B // nw
    ch = 128
    n_ch = b_per_w // ch
    mesh = plsc.VectorSubcoreMesh(core_axis_name="c", subcore_axis_name="s")

    @functools.partial(
        pl.kernel,
        mesh=mesh,
        out_type=jax.ShapeDtypeStruct((B, D), jnp.float32),
        scratch_types=[
            pltpu.VMEM((b_per_w,), jnp.int32),
            pltpu.VMEM((b_per_w, D), jnp.float32),
            pltpu.SemaphoreType.DMA,
        ],
    )
    def gather_k(table_hbm, idx_hbm, out_hbm, idx_v, rows_v, sem):
        wid = lax.axis_index("s") * info.num_cores + lax.axis_index("c")
        base = wid * b_per_w
        pltpu.sync_copy(idx_hbm.at[pl.ds(base, b_per_w)], idx_v)
        copies = []
        for j in range(n_ch):
            copies.append(
                pltpu.async_copy(
                    table_hbm.at[idx_v.at[pl.ds(j * ch, ch)]],
                    rows_v.at[pl.ds(j * ch, ch)],
                    sem,
                )
            )
        for c in copies:
            c.wait()
        pltpu.sync_copy(rows_v, out_hbm.at[pl.ds(base, b_per_w)])

    return gather_k(x, idx)


def _fused_body(
    g_ref, we_ref, wq_ref, wk_ref, wv_ref, wo_ref, bsel_ref,
    sc_ref, dc_ref, sr_ref, dr_ref, g1_ref, b1_ref,
    seeds_ref, wq2_ref, wk2_ref, wv2_ref, wo2_ref, g2_ref, b2_ref,
    p_ref,
    e_s, qa_s, ka_s, va_s, h_s, vbar_s,
):
    i = pl.program_id(0)

    @pl.when(i == 0)
    def _proj():
        xs = g_ref[0:E, :]
        xd = g_ref[E : 2 * E, :]
        e = jnp.dot(xs, we_ref[0:D, :], preferred_element_type=jnp.float32)
        e = e + jnp.dot(xd, we_ref[D : 2 * D, :], preferred_element_type=jnp.float32)
        e_s[...] = e
        q = jnp.dot(e, wq_ref[...], preferred_element_type=jnp.float32) * _SCALE2
        k = jnp.dot(e, wk_ref[...], preferred_element_type=jnp.float32)
        v = jnp.dot(e, wv_ref[...], preferred_element_type=jnp.float32)
        vbar_s[...] = jnp.mean(v, axis=0, keepdims=True)
        qn2 = jnp.dot(q * q, bsel_ref[...], preferred_element_type=jnp.float32)
        kn2 = jnp.dot(k * k, bsel_ref[...], preferred_element_type=jnp.float32)
        km = jnp.max(kn2, axis=0, keepdims=True)
        mh = jnp.sqrt(qn2 * km)
        ones = jnp.ones((E, 1), jnp.float32)
        zpad = jnp.zeros((E, 2 * HD - (HD + 1)), jnp.float32)
        for hh in range(H):
            sl = slice(hh * HD, (hh + 1) * HD)
            qa_s[hh] = jnp.concatenate([q[:, sl], mh[:, hh : hh + 1], zpad], axis=1)
            ka_s[hh] = jnp.concatenate([k[:, sl], -ones, zpad], axis=1)
            va_s[hh] = jnp.concatenate([v[:, sl], ones, zpad], axis=1).astype(
                jnp.bfloat16
            )

    sc = sc_ref[...]
    dc = dc_ref[...]
    row_ids = lax.broadcasted_iota(jnp.int32, (BQ, 1), 0) + i * BQ

    qas = [qa_s[hh, pl.ds(i * BQ, BQ), :] for hh in range(H)]
    accs = [jnp.zeros((BQ, 2 * HD), jnp.float32) for _ in range(H)]
    for kb in range(KN):
        sr = sr_ref[0:1, pl.ds(kb * KB, KB)]
        dr = dr_ref[0:1, pl.ds(kb * KB, KB)]
        adj = (sc == sr) | (dc == dr) | (sc == dr) | (dc == sr)
        col_ids = lax.broadcasted_iota(jnp.int32, (1, KB), 1) + kb * KB
        adjf = ((row_ids != col_ids) & adj).astype(jnp.bfloat16)
        for hh in range(H):
            kac = ka_s[hh, pl.ds(kb * KB, KB), :]
            s = jax.lax.dot_general(
                qas[hh], kac, (((1,), (1,)), ((), ())),
                preferred_element_type=jnp.float32,
            )
            p = jnp.exp2(s).astype(jnp.bfloat16) * adjf
            vac = va_s[hh, pl.ds(kb * KB, KB), :]
            accs[hh] = accs[hh] + jax.lax.dot_general(
                p, vac, (((1,), (0,)), ((), ())),
                preferred_element_type=jnp.float32,
            )

    outs = []
    for hh in range(H):
        oa = accs[hh]
        z = oa[:, HD : HD + 1]
        zs = jnp.where(z > 0.0, z, 1.0)
        o = oa[:, 0:HD] / zs
        outs.append(jnp.where(z > 0.0, o, vbar_s[:, hh * HD : (hh + 1) * HD]))
    o = jnp.concatenate(outs, axis=-1)
    hb = e_s[pl.ds(i * BQ, BQ), :] + jnp.dot(
        o, wo_ref[...], preferred_element_type=jnp.float32
    )
    mu = jnp.mean(hb, axis=-1, keepdims=True)
    var = jnp.mean((hb - mu) ** 2, axis=-1, keepdims=True)
    h_s[pl.ds(i * BQ, BQ), :] = (hb - mu) / jnp.sqrt(var + _EPS) * g1_ref[
        ...
    ] + b1_ref[...]

    @pl.when(i == NB - 1)
    def _pma():
        seeds = seeds_ref[...]
        hm = h_s[...]
        q2 = jnp.dot(seeds, wq2_ref[...], preferred_element_type=jnp.float32) * _SCALE
        k2 = jnp.dot(hm, wk2_ref[...], preferred_element_type=jnp.float32)
        v2 = jnp.dot(hm, wv2_ref[...], preferred_element_type=jnp.float32)
        outs2 = []
        for hh in range(H):
            qh = q2[:, hh * HD : (hh + 1) * HD]
            kh = k2[:, hh * HD : (hh + 1) * HD]
            vh = v2[:, hh * HD : (hh + 1) * HD]
            s = jax.lax.dot_general(
                qh, kh, (((1,), (1,)), ((), ())), preferred_element_type=jnp.float32
            )
            m = jnp.max(s, axis=-1, keepdims=True)
            p = jnp.exp(s - m)
            z = jnp.sum(p, axis=-1, keepdims=True)
            outs2.append(jnp.dot(p, vh, preferred_element_type=jnp.float32) / z)
        o2 = jnp.concatenate(outs2, axis=-1)
        pb = seeds + jnp.dot(o2, wo2_ref[...], preferred_element_type=jnp.float32)
        mu2 = jnp.mean(pb, axis=-1, keepdims=True)
        var2 = jnp.mean((pb - mu2) ** 2, axis=-1, keepdims=True)
        p_ref[...] = (pb - mu2) / jnp.sqrt(var2 + _EPS) * g2_ref[...] + b2_ref[...]


def _tc_pipeline(g, edge_index, W_e, Wq, Wk, Wv, Wo, seeds,
                 Wq2, Wk2, Wv2, Wo2, g1, b1, g2, b2, interpret=False):
    f32 = jnp.float32
    src_c = edge_index[0].reshape(E, 1)
    dst_c = edge_index[1].reshape(E, 1)
    src_r = edge_index[0].reshape(1, E)
    dst_r = edge_index[1].reshape(1, E)
    bsel = jnp.asarray(
        np.repeat(np.eye(H, dtype=np.float32), HD, axis=0)
    )
    bsel = jnp.pad(bsel, ((0, 0), (0, DH - H)))
    full = lambda shape: pl.BlockSpec(shape, lambda i: (0, 0))
    p = pl.pallas_call(
        _fused_body,
        grid=(NB,),
        in_specs=[
            full((2 * E, D)),
            full((2 * D, DH)),
            full((DH, DH)),
            full((DH, DH)),
            full((DH, DH)),
            full((DH, DH)),
            full((DH, DH)),
            pl.BlockSpec((BQ, 1), lambda i: (i, 0)),
            pl.BlockSpec((BQ, 1), lambda i: (i, 0)),
            full((1, E)),
            full((1, E)),
            full((1, DH)),
            full((1, DH)),
            full((S, DH)),
            full((DH, DH)),
            full((DH, DH)),
            full((DH, DH)),
            full((DH, DH)),
            full((1, DH)),
            full((1, DH)),
        ],
        out_specs=pl.BlockSpec((S, DH), lambda i: (0, 0)),
        out_shape=jax.ShapeDtypeStruct((S, DH), f32),
        scratch_shapes=[
            pltpu.VMEM((E, DH), f32),
            pltpu.VMEM((H, E, 2 * HD), f32),
            pltpu.VMEM((H, E, 2 * HD), f32),
            pltpu.VMEM((H, E, 2 * HD), jnp.bfloat16),
            pltpu.VMEM((E, DH), f32),
            pltpu.VMEM((1, DH), f32),
        ],
        interpret=interpret,
    )(
        g, W_e, Wq, Wk, Wv, Wo, bsel, src_c, dst_c, src_r, dst_r,
        g1.reshape(1, DH), b1.reshape(1, DH), seeds,
        Wq2, Wk2, Wv2, Wo2, g2.reshape(1, DH), b2.reshape(1, DH),
    )
    return p


def kernel(x, edge_index, W_e, Wq, Wk, Wv, Wo, seeds, Wq2, Wk2, Wv2, Wo2, g1, b1, g2, b2):
    idx = edge_index.reshape(2 * E)
    g = _sc_gather(x, idx)
    return _tc_pipeline(g, edge_index, W_e, Wq, Wk, Wv, Wo, seeds,
                        Wq2, Wk2, Wv2, Wo2, g1, b1, g2, b2)


# speedup vs baseline: 1.0408x; 1.0009x over previous
import functools

import jax
import jax.numpy as jnp
import numpy as np
from jax import lax
from jax.experimental import pallas as pl
from jax.experimental.pallas import tpu as pltpu
from jax.experimental.pallas import tpu_sc as plsc

N = 8192
E = 4096
D = 128
DH = 128
H = 4
HD = DH // H
S = 32

_SCALE = float(1.0 / np.sqrt(HD))
_SCALE2 = float(1.0 / (np.sqrt(HD) * np.log(2.0)))
_NEG = -1e9
_EPS = 1e-5

BQ = 512
NB = E // BQ
KB = 512
KN = E // KB


def _sc_gather(x, idx):
    B = idx.shape[0]
    info = plsc.get_sparse_core_info()
    nw = info.num_cores * info.num_subcores
    b_per_w = B // nw
    ch = 128
    n_ch = b_per_w // ch
    mesh = plsc.VectorSubcoreMesh(core_axis_name="c", subcore_axis_name="s")

    @functools.partial(
        pl.kernel,
        mesh=mesh,
        out_type=jax.ShapeDtypeStruct((B, D), jnp.float32),
        scratch_types=[
            pltpu.VMEM((b_per_w,), jnp.int32),
            pltpu.VMEM((b_per_w, D), jnp.float32),
            pltpu.SemaphoreType.DMA,
        ],
    )
    def gather_k(table_hbm, idx_hbm, out_hbm, idx_v, rows_v, sem):
        wid = lax.axis_index("s") * info.num_cores + lax.axis_index("c")
        base = wid * b_per_w
        pltpu.sync_copy(idx_hbm.at[pl.ds(base, b_per_w)], idx_v)
        copies = []
        for j in range(n_ch):
            copies.append(
                pltpu.async_copy(
                    table_hbm.at[idx_v.at[pl.ds(j * ch, ch)]],
                    rows_v.at[pl.ds(j * ch, ch)],
                    sem,
                )
            )
        for c in copies:
            c.wait()
        pltpu.sync_copy(rows_v, out_hbm.at[pl.ds(base, b_per_w)])

    return gather_k(x, idx)


def _fused_body(
    g_ref, we_ref, wq_ref, wk_ref, wv_ref, wo_ref, bsel_ref,
    sc_ref, dc_ref, sr_ref, dr_ref, g1_ref, b1_ref,
    seeds_ref, wq2_ref, wk2_ref, wv2_ref, wo2_ref, g2_ref, b2_ref,
    p_ref,
    e_s, qa_s, ka_s, va_s, h_s, vbar_s,
):
    i = pl.program_id(0)

    @pl.when(i == 0)
    def _proj():
        xs = g_ref[0:E, :]
        xd = g_ref[E : 2 * E, :]
        e = jnp.dot(xs, we_ref[0:D, :], preferred_element_type=jnp.float32)
        e = e + jnp.dot(xd, we_ref[D : 2 * D, :], preferred_element_type=jnp.float32)
        e_s[...] = e
        q = jnp.dot(e, wq_ref[...], preferred_element_type=jnp.float32) * _SCALE2
        k = jnp.dot(e, wk_ref[...], preferred_element_type=jnp.float32)
        v = jnp.dot(e, wv_ref[...], preferred_element_type=jnp.float32)
        vbar_s[...] = jnp.mean(v, axis=0, keepdims=True)
        qn2 = jnp.dot(q * q, bsel_ref[...], preferred_element_type=jnp.float32)
        kn2 = jnp.dot(k * k, bsel_ref[...], preferred_element_type=jnp.float32)
        km = jnp.max(kn2, axis=0, keepdims=True)
        mh = jnp.sqrt(qn2 * km)
        ones = jnp.ones((E, 1), jnp.float32)
        zpad = jnp.zeros((E, 2 * HD - (HD + 1)), jnp.float32)
        for hh in range(H):
            sl = slice(hh * HD, (hh + 1) * HD)
            qa_s[hh] = jnp.concatenate([q[:, sl], mh[:, hh : hh + 1], zpad], axis=1)
            ka_s[hh] = jnp.concatenate([k[:, sl], -ones, zpad], axis=1)
            va_s[hh] = jnp.concatenate([v[:, sl], ones, zpad], axis=1).astype(
                jnp.bfloat16
            )

    sc = sc_ref[...]
    dc = dc_ref[...]
    row_ids = lax.broadcasted_iota(jnp.int32, (BQ, 1), 0) + i * BQ

    qas = [qa_s[hh, pl.ds(i * BQ, BQ), :] for hh in range(H)]
    accs = [jnp.zeros((BQ, 2 * HD), jnp.float32) for _ in range(H)]
    for kb in range(KN):
        sr = sr_ref[0:1, pl.ds(kb * KB, KB)]
        dr = dr_ref[0:1, pl.ds(kb * KB, KB)]
        adj = (sc == sr) | (dc == dr) | (sc == dr) | (dc == sr)
        col_ids = lax.broadcasted_iota(jnp.int32, (1, KB), 1) + kb * KB
        adjf = ((row_ids != col_ids) & adj).astype(jnp.bfloat16)
        for hh in range(H):
            kac = ka_s[hh, pl.ds(kb * KB, KB), :]
            s = jax.lax.dot_general(
                qas[hh], kac, (((1,), (1,)), ((), ())),
                preferred_element_type=jnp.float32,
            )
            p = jnp.exp2(s).astype(jnp.bfloat16) * adjf
            vac = va_s[hh, pl.ds(kb * KB, KB), :]
            accs[hh] = accs[hh] + jax.lax.dot_general(
                p, vac, (((1,), (0,)), ((), ())),
                preferred_element_type=jnp.float32,
            )

    outs = []
    for hh in range(H):
        oa = accs[hh]
        z = oa[:, HD : HD + 1]
        zs = jnp.where(z > 0.0, z, 1.0)
        o = oa[:, 0:HD] / zs
        outs.append(jnp.where(z > 0.0, o, vbar_s[:, hh * HD : (hh + 1) * HD]))
    o = jnp.concatenate(outs, axis=-1)
    hb = e_s[pl.ds(i * BQ, BQ), :] + jnp.dot(
        o, wo_ref[...], preferred_element_type=jnp.float32
    )
    mu = jnp.mean(hb, axis=-1, keepdims=True)
    var = jnp.mean((hb - mu) ** 2, axis=-1, keepdims=True)
    h_s[pl.ds(i * BQ, BQ), :] = (hb - mu) / jnp.sqrt(var + _EPS) * g1_ref[
        ...
    ] + b1_ref[...]

    @pl.when(i == NB - 1)
    def _pma():
        seeds = seeds_ref[...]
        hm = h_s[...]
        q2 = jnp.dot(seeds, wq2_ref[...], preferred_element_type=jnp.float32) * _SCALE
        k2 = jnp.dot(hm, wk2_ref[...], preferred_element_type=jnp.float32)
        v2 = jnp.dot(hm, wv2_ref[...], preferred_element_type=jnp.float32)
        outs2 = []
        for hh in range(H):
            qh = q2[:, hh * HD : (hh + 1) * HD]
            kh = k2[:, hh * HD : (hh + 1) * HD]
            vh = v2[:, hh * HD : (hh + 1) * HD]
            s = jax.lax.dot_general(
                qh, kh, (((1,), (1,)), ((), ())), preferred_element_type=jnp.float32
            )
            m = jnp.max(s, axis=-1, keepdims=True)
            p = jnp.exp(s - m)
            z = jnp.sum(p, axis=-1, keepdims=True)
            outs2.append(jnp.dot(p, vh, preferred_element_type=jnp.float32) / z)
        o2 = jnp.concatenate(outs2, axis=-1)
        pb = seeds + jnp.dot(o2, wo2_ref[...], preferred_element_type=jnp.float32)
        mu2 = jnp.mean(pb, axis=-1, keepdims=True)
        var2 = jnp.mean((pb - mu2) ** 2, axis=-1, keepdims=True)
        p_ref[...] = (pb - mu2) / jnp.sqrt(var2 + _EPS) * g2_ref[...] + b2_ref[...]


def _tc_pipeline(g, edge_index, W_e, Wq, Wk, Wv, Wo, seeds,
                 Wq2, Wk2, Wv2, Wo2, g1, b1, g2, b2, interpret=False):
    f32 = jnp.float32
    src_c = edge_index[0].reshape(E, 1)
    dst_c = edge_index[1].reshape(E, 1)
    src_r = edge_index[0].reshape(1, E)
    dst_r = edge_index[1].reshape(1, E)
    bsel = jnp.asarray(
        np.repeat(np.eye(H, dtype=np.float32), HD, axis=0)
    )
    bsel = jnp.pad(bsel, ((0, 0), (0, DH - H)))
    full = lambda shape: pl.BlockSpec(shape, lambda i: (0, 0))
    p = pl.pallas_call(
        _fused_body,
        grid=(NB,),
        in_specs=[
            full((2 * E, D)),
            full((2 * D, DH)),
            full((DH, DH)),
            full((DH, DH)),
            full((DH, DH)),
            full((DH, DH)),
            full((DH, DH)),
            pl.BlockSpec((BQ, 1), lambda i: (i, 0)),
            pl.BlockSpec((BQ, 1), lambda i: (i, 0)),
            full((1, E)),
            full((1, E)),
            full((1, DH)),
            full((1, DH)),
            full((S, DH)),
            full((DH, DH)),
            full((DH, DH)),
            full((DH, DH)),
            full((DH, DH)),
            full((1, DH)),
            full((1, DH)),
        ],
        out_specs=pl.BlockSpec((S, DH), lambda i: (0, 0)),
        out_shape=jax.ShapeDtypeStruct((S, DH), f32),
        scratch_shapes=[
            pltpu.VMEM((E, DH), f32),
            pltpu.VMEM((H, E, 2 * HD), f32),
            pltpu.VMEM((H, E, 2 * HD), f32),
            pltpu.VMEM((H, E, 2 * HD), jnp.bfloat16),
            pltpu.VMEM((E, DH), f32),
            pltpu.VMEM((1, DH), f32),
        ],
        interpret=interpret,
    )(
        g, W_e, Wq, Wk, Wv, Wo, bsel, src_c, dst_c, src_r, dst_r,
        g1.reshape(1, DH), b1.reshape(1, DH), seeds,
        Wq2, Wk2, Wv2, Wo2, g2.reshape(1, DH), b2.reshape(1, DH),
    )
    return p


def kernel(x, edge_index, W_e, Wq, Wk, Wv, Wo, seeds, Wq2, Wk2, Wv2, Wo2, g1, b1, g2, b2):
    idx = edge_index.reshape(2 * E)
    g = _sc_gather(x, idx)
    return _tc_pipeline(g, edge_index, W_e, Wq, Wk, Wv, Wo, seeds,
                        Wq2, Wk2, Wv2, Wo2, g1, b1, g2, b2)
